# baseline (speedup 1.0000x reference)
import functools

import jax
import jax.numpy as jnp
from jax import lax
from jax.experimental import pallas as pl
from jax.experimental.pallas import tpu as pltpu
from jax.experimental.pallas import tpu_sc as plsc

R, C = 16384, 128
NC, NS, L = 2, 16, 16
NW = NC * NS
SCROWS = 4096
RPW = SCROWS // NW
CG = C // L
CH = 64
NCH = RPW // CH
RB = 2048
BR = 4096

_mesh = plsc.VectorSubcoreMesh(core_axis_name="c", subcore_axis_name="s")


@functools.partial(
    pl.kernel,
    mesh=_mesh,
    out_type=jax.ShapeDtypeStruct((NW, C), jnp.float32),
    scratch_types=[
        pltpu.VMEM((CH, C), jnp.float32),
        pltpu.VMEM((CH, C), jnp.float32),
        pltpu.VMEM((1, C), jnp.float32),
        pltpu.SemaphoreType.DMA,
        pltpu.SemaphoreType.DMA,
    ],
)
def _sc_partial_sums(x_hbm, out_hbm, xb0, xb1, accbuf, sem0, sem1):
    wid = lax.axis_index("s") * NC + lax.axis_index("c")
    base = wid * RPW
    bufs = (xb0, xb1)
    sems = (sem0, sem1)

    cps = [
        pltpu.async_copy(x_hbm.at[pl.ds(base + ch * CH, CH)], bufs[ch], sems[ch])
        for ch in range(NCH)
    ]
    accs = tuple(jnp.zeros((L,), jnp.float32) for _ in range(CG))
    for ch in range(NCH):
        cps[ch].wait()
        buf = bufs[ch]

        def body(r, accs, buf=buf):
            return tuple(accs[g] + buf[r, pl.ds(g * L, L)] for g in range(CG))

        accs = lax.fori_loop(0, CH, body, accs, unroll=4)
    for g in range(CG):
        accbuf[0, pl.ds(g * L, L)] = accs[g]
    pltpu.sync_copy(accbuf, out_hbm.at[pl.ds(wid, 1)])


ARMS = 2
TCSTEPS = (R - SCROWS) // (ARMS * RB)


def _tc_reduce_body(*refs):
    o_ref = refs[ARMS]
    part = refs[0][...].sum(axis=0, keepdims=True)
    for a in range(1, ARMS):
        part = part + refs[a][...].sum(axis=0, keepdims=True)

    @pl.when(pl.program_id(0) == 0)
    def _():
        o_ref[...] = part

    @pl.when(pl.program_id(0) > 0)
    def _():
        o_ref[...] += part


def _tc_partial_sums(x):
    base = SCROWS // RB
    in_specs = [
        pl.BlockSpec((RB, C), lambda i, a=a: (base + a * TCSTEPS + i, 0))
        for a in range(ARMS)
    ]
    return pl.pallas_call(
        _tc_reduce_body,
        grid=(TCSTEPS,),
        in_specs=in_specs,
        out_specs=pl.BlockSpec((1, C), lambda i: (0, 0)),
        out_shape=jax.ShapeDtypeStruct((1, C), jnp.float32),
    )(*([x] * ARMS))


def _tc_choice_body(ps_ref, pt_ref, o_ref, v_ref):
    @pl.when(pl.program_id(0) == 0)
    def _():
        s = jnp.sum(ps_ref[...], axis=0, keepdims=True) + pt_ref[...]
        iota = lax.broadcasted_iota(jnp.int32, (1, C), 1)
        cand = jnp.where(s >= 0.0, iota, jnp.int32(C))
        idx = jnp.min(cand)
        idx = jnp.where(idx >= C, jnp.int32(0), idx)
        v_ref[...] = jnp.where(iota == idx, 1.0, -1.0).astype(jnp.float32)

    o_ref[...] = jnp.broadcast_to(v_ref[...], (BR, C))


def _tc_broadcast_choice(ps_sc, ps_tc):
    return pl.pallas_call(
        _tc_choice_body,
        grid=(R // BR,),
        in_specs=[
            pl.BlockSpec((NW, C), lambda i: (0, 0)),
            pl.BlockSpec((1, C), lambda i: (0, 0)),
        ],
        out_specs=pl.BlockSpec((BR, C), lambda i: (i, 0)),
        out_shape=jax.ShapeDtypeStruct((R, C), jnp.float32),
        scratch_shapes=[pltpu.VMEM((1, C), jnp.float32)],
    )(ps_sc, ps_tc)


def kernel(x):
    ps_tc = _tc_partial_sums(x)
    ps_sc = _sc_partial_sums(x)
    return _tc_broadcast_choice(ps_sc, ps_tc)


# speedup vs baseline: 1.0058x; 1.0058x over previous
import functools

import jax
import jax.numpy as jnp
from jax import lax
from jax.experimental import pallas as pl
from jax.experimental.pallas import tpu as pltpu
from jax.experimental.pallas import tpu_sc as plsc

R, C = 16384, 128
NC, NS, L = 2, 16, 16
NW = NC * NS
SCROWS = 4096
RPW = SCROWS // NW
CG = C // L
CH = 64
NCH = RPW // CH
RB = 2048
BR = 4096

_mesh = plsc.VectorSubcoreMesh(core_axis_name="c", subcore_axis_name="s")


@functools.partial(
    pl.kernel,
    mesh=_mesh,
    out_type=jax.ShapeDtypeStruct((NW, C), jnp.float32),
    scratch_types=[
        pltpu.VMEM((CH, C), jnp.float32),
        pltpu.VMEM((CH, C), jnp.float32),
        pltpu.VMEM((1, C), jnp.float32),
        pltpu.SemaphoreType.DMA,
        pltpu.SemaphoreType.DMA,
    ],
)
def _sc_partial_sums(x_hbm, out_hbm, xb0, xb1, accbuf, sem0, sem1):
    wid = lax.axis_index("s") * NC + lax.axis_index("c")
    base = wid * RPW
    bufs = (xb0, xb1)
    sems = (sem0, sem1)

    cps = [
        pltpu.async_copy(x_hbm.at[pl.ds(base + ch * CH, CH)], bufs[ch], sems[ch])
        for ch in range(NCH)
    ]
    accs = tuple(jnp.zeros((L,), jnp.float32) for _ in range(CG))
    for ch in range(NCH):
        cps[ch].wait()
        buf = bufs[ch]

        def body(r, accs, buf=buf):
            return tuple(accs[g] + buf[r, pl.ds(g * L, L)] for g in range(CG))

        accs = lax.fori_loop(0, CH, body, accs, unroll=4)
    for g in range(CG):
        accbuf[0, pl.ds(g * L, L)] = accs[g]
    pltpu.sync_copy(accbuf, out_hbm.at[pl.ds(wid, 1)])


def _tc_reduce_body(x_ref, o_ref):
    part = jnp.sum(x_ref[...], axis=0, keepdims=True)

    @pl.when(pl.program_id(0) == 0)
    def _():
        o_ref[...] = part

    @pl.when(pl.program_id(0) > 0)
    def _():
        o_ref[...] += part


def _tc_partial_sums(x):
    return pl.pallas_call(
        _tc_reduce_body,
        grid=((R - SCROWS) // RB,),
        in_specs=[pl.BlockSpec((RB, C), lambda i: (SCROWS // RB + i, 0))],
        out_specs=pl.BlockSpec((1, C), lambda i: (0, 0)),
        out_shape=jax.ShapeDtypeStruct((1, C), jnp.float32),
    )(x)


def _tc_choice_body(ps_ref, pt_ref, o_ref, v_ref):
    @pl.when(pl.program_id(0) == 0)
    def _():
        s = jnp.sum(ps_ref[...], axis=0, keepdims=True) + pt_ref[...]
        iota = lax.broadcasted_iota(jnp.int32, (1, C), 1)
        cand = jnp.where(s >= 0.0, iota, jnp.int32(C))
        idx = jnp.min(cand)
        idx = jnp.where(idx >= C, jnp.int32(0), idx)
        v_ref[...] = jnp.where(iota == idx, 1.0, -1.0).astype(jnp.float32)

    o_ref[...] = jnp.broadcast_to(v_ref[...], (BR, C))


def _tc_broadcast_choice(ps_sc, ps_tc):
    return pl.pallas_call(
        _tc_choice_body,
        grid=(R // BR,),
        in_specs=[
            pl.BlockSpec((NW, C), lambda i: (0, 0)),
            pl.BlockSpec((1, C), lambda i: (0, 0)),
        ],
        out_specs=pl.BlockSpec((BR, C), lambda i: (i, 0)),
        out_shape=jax.ShapeDtypeStruct((R, C), jnp.float32),
        scratch_shapes=[pltpu.VMEM((1, C), jnp.float32)],
    )(ps_sc, ps_tc)


def kernel(x):
    ps_tc = _tc_partial_sums(x)
    ps_sc = _sc_partial_sums(x)
    return _tc_broadcast_choice(ps_sc, ps_tc)
